# baseline (speedup 1.0000x reference)
import jax
import jax.numpy as jnp
from jax.experimental import pallas as pl
from jax.experimental.pallas import tpu as pltpu

_BF16 = jnp.bfloat16
_F32 = jnp.float32


def _half_net(x_ref, t1_ref, b1_ref, t2_ref, b2_ref,
              w1f_ref, b1f_ref, w2f_ref, b2f_ref, o_ref,
              a1_ref, a2_ref, h, Bh):
    lo = h * Bh

    for r in range(4):
        for di in range(5):
            q, o = (r + di) % 4, (r + di) // 4
            a1_ref[r, :, :, 68 * di:68 * di + 68] = \
                x_ref[lo:lo + Bh, o:o + 16, 68 * q:68 * q + 68]

    y1 = jnp.dot(a1_ref[...].reshape(4 * Bh * 16, 340), t1_ref[...],
                 preferred_element_type=_F32).reshape(4, Bh, 16, 768)
    b1v = b1_ref[...]
    pe = jnp.maximum(jnp.maximum(y1[0, :, :, :384], y1[0, :, :, 384:]),
                     jnp.maximum(y1[1, :, :, :384], y1[1, :, :, 384:]))
    po = jnp.maximum(jnp.maximum(y1[2, :, :, :384], y1[2, :, :, 384:]),
                     jnp.maximum(y1[3, :, :, :384], y1[3, :, :, 384:]))
    pe = jnp.maximum(pe + b1v, 0.0).astype(_BF16)
    po = jnp.maximum(po + b1v, 0.0).astype(_BF16)

    zrow = jnp.zeros((Bh, 1, 384), _BF16)
    shifted = {
        (0, 0): jnp.concatenate([zrow, pe[:, 0:15, :]], axis=1),
        (0, 1): pe,
        (0, 2): jnp.concatenate([pe[:, 1:16, :], zrow], axis=1),
        (1, 0): jnp.concatenate([zrow, po[:, 0:15, :]], axis=1),
        (1, 1): po,
        (1, 2): jnp.concatenate([po[:, 1:16, :], zrow], axis=1),
    }
    for s in range(2):
        for di in range(5):
            par, off = (s + di) % 2, (s + di) // 2
            a2_ref[s, :, :, 384 * di:384 * di + 384] = shifted[(par, off)]

    y2 = jnp.dot(a2_ref[...].reshape(2 * Bh * 16, 1920), t2_ref[...],
                 preferred_element_type=_F32).reshape(2, Bh, 16, 640)
    b2v = b2_ref[...]
    f = jnp.maximum(jnp.maximum(y2[0, :, :, :320], y2[0, :, :, 320:]),
                    jnp.maximum(y2[1, :, :, :320], y2[1, :, :, 320:]))
    f = jnp.maximum(f + b2v, 0.0).astype(_BF16)

    parts = [jnp.dot(f[:, i, :], w1f_ref[i], preferred_element_type=_F32)
             for i in range(16)]
    while len(parts) > 1:
        parts = [parts[j] + parts[j + 1] for j in range(0, len(parts), 2)]
    hacc = jnp.maximum(parts[0] + b1f_ref[...], 0.0).astype(_BF16)
    o_ref[lo:lo + Bh, :] = jnp.dot(hacc, w2f_ref[...],
                                   preferred_element_type=_F32) + b2f_ref[...]


def _net_kernel(x_ref, t1_ref, b1_ref, t2_ref, b2_ref,
                w1f_ref, b1f_ref, w2f_ref, b2f_ref, o_ref,
                a1_ref0, a2_ref0, a1_ref1, a2_ref1):
    B = x_ref.shape[0]
    Bh = a1_ref0.shape[1]
    _half_net(x_ref, t1_ref, b1_ref, t2_ref, b2_ref,
              w1f_ref, b1f_ref, w2f_ref, b2f_ref, o_ref,
              a1_ref0, a2_ref0, 0, Bh)
    if B > Bh:
        _half_net(x_ref, t1_ref, b1_ref, t2_ref, b2_ref,
                  w1f_ref, b1f_ref, w2f_ref, b2f_ref, o_ref,
                  a1_ref1, a2_ref1, 1, Bh)


def _build_t1(w1, b1):
    w1r = w1.reshape(10, 5, 5)
    d5 = (jnp.arange(68)[None, :, None]
          == jnp.arange(64)[None, None, :] + jnp.arange(5)[:, None, None])
    t1 = jnp.einsum('cie,etj->itcj', w1r, d5.astype(_F32))
    t1e = jnp.pad(t1[..., 0::2], ((0, 0), (0, 0), (0, 0), (2, 2)))
    t1o = jnp.pad(t1[..., 1::2], ((0, 0), (0, 0), (0, 0), (2, 2)))
    t1e = jnp.pad(t1e.reshape(340, 360), ((0, 0), (0, 24)))
    t1o = jnp.pad(t1o.reshape(340, 360), ((0, 0), (0, 24)))
    t1m = jnp.concatenate([t1e, t1o], axis=1).astype(_BF16)
    b1c = jnp.pad(jnp.broadcast_to(b1, (10, 32)),
                  ((0, 0), (2, 2))).reshape(1, 360)
    b1c = jnp.pad(b1c, ((0, 0), (0, 24)))
    return t1m, b1c


def _build_t2(w2, b2):
    w2r = w2.reshape(20, 5, 5, 10)
    d5 = (jnp.arange(36)[None, :, None]
          == jnp.arange(32)[None, None, :] + jnp.arange(5)[:, None, None])
    t2 = jnp.einsum('diec,etj->ictdj', w2r, d5.astype(_F32))
    t2e = jnp.pad(t2[..., 0::2].reshape(5, 360, 320), ((0, 0), (0, 24), (0, 0)))
    t2o = jnp.pad(t2[..., 1::2].reshape(5, 360, 320), ((0, 0), (0, 24), (0, 0)))
    t2m = jnp.concatenate([t2e.reshape(1920, 320),
                           t2o.reshape(1920, 320)],
                          axis=1).astype(_BF16)
    b2v = jnp.broadcast_to(b2, (20, 16)).reshape(1, 320)
    return t2m, b2v


def kernel(x, w1, b1, w2, b2, fc1_w, fc1_b, fc2_w, fc2_b):
    N = x.shape[0]
    B = 64
    while N % B:
        B //= 2
    Bh = max(B // 2, 1)

    xpad = jnp.pad(x[:, 0], ((0, 0), (2, 2), (2, 2)))
    x4 = xpad.reshape(N, 17, 272).astype(_BF16)

    t1m, b1c = _build_t1(w1, b1)
    t2m, b2v = _build_t2(w2, b2)
    fc1_ws = fc1_w.reshape(20, 16, 16, 128).transpose(1, 0, 2, 3) \
        .reshape(16, 320, 128).astype(_BF16)
    fc2_wb = fc2_w.astype(_BF16)
    n_out = fc2_w.shape[1]

    return pl.pallas_call(
        _net_kernel,
        out_shape=jax.ShapeDtypeStruct((N, n_out), _F32),
        grid=(N // B,),
        in_specs=[
            pl.BlockSpec((B, 17, 272), lambda n: (n, 0, 0)),
            pl.BlockSpec((340, 768), lambda n: (0, 0)),
            pl.BlockSpec((1, 384), lambda n: (0, 0)),
            pl.BlockSpec((1920, 640), lambda n: (0, 0)),
            pl.BlockSpec((1, 320), lambda n: (0, 0)),
            pl.BlockSpec((16, 320, 128), lambda n: (0, 0, 0)),
            pl.BlockSpec((1, 128), lambda n: (0, 0)),
            pl.BlockSpec((128, n_out), lambda n: (0, 0)),
            pl.BlockSpec((1, n_out), lambda n: (0, 0)),
        ],
        out_specs=pl.BlockSpec((B, n_out), lambda n: (n, 0)),
        scratch_shapes=[
            pltpu.VMEM((4, Bh, 16, 340), _BF16),
            pltpu.VMEM((2, Bh, 16, 1920), _BF16),
            pltpu.VMEM((4, Bh, 16, 340), _BF16),
            pltpu.VMEM((2, Bh, 16, 1920), _BF16),
        ],
        compiler_params=pltpu.CompilerParams(
            dimension_semantics=("parallel",)),
    )(x4, t1m, b1c, t2m, b2v, fc1_ws, fc1_b, fc2_wb, fc2_b)


# speedup vs baseline: 1.0758x; 1.0758x over previous
import jax
import jax.numpy as jnp
from jax.experimental import pallas as pl
from jax.experimental.pallas import tpu as pltpu

_BF16 = jnp.bfloat16
_F32 = jnp.float32


def _half_net(x_ref, t1_ref, b1_ref, t2_ref, b2_ref,
              a1_ref, a2_ref, h, Bh):
    lo = h * Bh

    for r in range(4):
        for di in range(5):
            q, o = (r + di) % 4, (r + di) // 4
            a1_ref[r, :, :, 68 * di:68 * di + 68] = \
                x_ref[lo:lo + Bh, o:o + 16, 68 * q:68 * q + 68]

    y1 = jnp.dot(a1_ref[...].reshape(4 * Bh * 16, 340), t1_ref[...],
                 preferred_element_type=_F32).reshape(4, Bh, 16, 768)
    b1v = b1_ref[...]
    pe = jnp.maximum(jnp.maximum(y1[0, :, :, :384], y1[0, :, :, 384:]),
                     jnp.maximum(y1[1, :, :, :384], y1[1, :, :, 384:]))
    po = jnp.maximum(jnp.maximum(y1[2, :, :, :384], y1[2, :, :, 384:]),
                     jnp.maximum(y1[3, :, :, :384], y1[3, :, :, 384:]))
    pe = jnp.maximum(pe + b1v, 0.0).astype(_BF16)
    po = jnp.maximum(po + b1v, 0.0).astype(_BF16)

    zrow = jnp.zeros((Bh, 1, 384), _BF16)
    shifted = {
        (0, 0): jnp.concatenate([zrow, pe[:, 0:15, :]], axis=1),
        (0, 1): pe,
        (0, 2): jnp.concatenate([pe[:, 1:16, :], zrow], axis=1),
        (1, 0): jnp.concatenate([zrow, po[:, 0:15, :]], axis=1),
        (1, 1): po,
        (1, 2): jnp.concatenate([po[:, 1:16, :], zrow], axis=1),
    }
    for s in range(2):
        for di in range(5):
            par, off = (s + di) % 2, (s + di) // 2
            a2_ref[s, :, :, 384 * di:384 * di + 384] = shifted[(par, off)]

    y2 = jnp.dot(a2_ref[...].reshape(2 * Bh * 16, 1920), t2_ref[...],
                 preferred_element_type=_F32).reshape(2, Bh, 16, 640)
    b2v = b2_ref[...]
    f = jnp.maximum(jnp.maximum(y2[0, :, :, :320], y2[0, :, :, 320:]),
                    jnp.maximum(y2[1, :, :, :320], y2[1, :, :, 320:]))
    return jnp.maximum(f + b2v, 0.0).astype(_BF16)


def _net_kernel(x_ref, t1_ref, b1_ref, t2_ref, b2_ref,
                w1f_ref, b1f_ref, w2f_ref, b2f_ref, o_ref,
                a1_ref0, a2_ref0, a1_ref1, a2_ref1):
    B = x_ref.shape[0]
    Bh = a1_ref0.shape[1]
    f = _half_net(x_ref, t1_ref, b1_ref, t2_ref, b2_ref,
                  a1_ref0, a2_ref0, 0, Bh)
    if B > Bh:
        f1 = _half_net(x_ref, t1_ref, b1_ref, t2_ref, b2_ref,
                       a1_ref1, a2_ref1, 1, Bh)
        f = jnp.concatenate([f, f1], axis=0)

    parts = [jnp.dot(f[:, i, :], w1f_ref[i], preferred_element_type=_F32)
             for i in range(16)]
    while len(parts) > 1:
        parts = [parts[j] + parts[j + 1] for j in range(0, len(parts), 2)]
    hacc = jnp.maximum(parts[0] + b1f_ref[...], 0.0).astype(_BF16)
    o_ref[...] = jnp.dot(hacc, w2f_ref[...],
                         preferred_element_type=_F32) + b2f_ref[...]


def _build_t1(w1, b1):
    w1r = w1.reshape(10, 5, 5)
    d5 = (jnp.arange(68)[None, :, None]
          == jnp.arange(64)[None, None, :] + jnp.arange(5)[:, None, None])
    t1 = jnp.einsum('cie,etj->itcj', w1r, d5.astype(_F32))
    t1e = jnp.pad(t1[..., 0::2], ((0, 0), (0, 0), (0, 0), (2, 2)))
    t1o = jnp.pad(t1[..., 1::2], ((0, 0), (0, 0), (0, 0), (2, 2)))
    t1e = jnp.pad(t1e.reshape(340, 360), ((0, 0), (0, 24)))
    t1o = jnp.pad(t1o.reshape(340, 360), ((0, 0), (0, 24)))
    t1m = jnp.concatenate([t1e, t1o], axis=1).astype(_BF16)
    b1c = jnp.pad(jnp.broadcast_to(b1, (10, 32)),
                  ((0, 0), (2, 2))).reshape(1, 360)
    b1c = jnp.pad(b1c, ((0, 0), (0, 24)))
    return t1m, b1c


def _build_t2(w2, b2):
    w2r = w2.reshape(20, 5, 5, 10)
    d5 = (jnp.arange(36)[None, :, None]
          == jnp.arange(32)[None, None, :] + jnp.arange(5)[:, None, None])
    t2 = jnp.einsum('diec,etj->ictdj', w2r, d5.astype(_F32))
    t2e = jnp.pad(t2[..., 0::2].reshape(5, 360, 320), ((0, 0), (0, 24), (0, 0)))
    t2o = jnp.pad(t2[..., 1::2].reshape(5, 360, 320), ((0, 0), (0, 24), (0, 0)))
    t2m = jnp.concatenate([t2e.reshape(1920, 320),
                           t2o.reshape(1920, 320)],
                          axis=1).astype(_BF16)
    b2v = jnp.broadcast_to(b2, (20, 16)).reshape(1, 320)
    return t2m, b2v


def kernel(x, w1, b1, w2, b2, fc1_w, fc1_b, fc2_w, fc2_b):
    N = x.shape[0]
    B = 64
    while N % B:
        B //= 2
    Bh = max(B // 2, 1)

    xpad = jnp.pad(x[:, 0], ((0, 0), (2, 2), (2, 2)))
    x4 = xpad.reshape(N, 17, 272).astype(_BF16)

    t1m, b1c = _build_t1(w1, b1)
    t2m, b2v = _build_t2(w2, b2)
    fc1_ws = fc1_w.reshape(20, 16, 16, 128).transpose(1, 0, 2, 3) \
        .reshape(16, 320, 128).astype(_BF16)
    fc2_wb = fc2_w.astype(_BF16)
    n_out = fc2_w.shape[1]

    return pl.pallas_call(
        _net_kernel,
        out_shape=jax.ShapeDtypeStruct((N, n_out), _F32),
        grid=(N // B,),
        in_specs=[
            pl.BlockSpec((B, 17, 272), lambda n: (n, 0, 0)),
            pl.BlockSpec((340, 768), lambda n: (0, 0)),
            pl.BlockSpec((1, 384), lambda n: (0, 0)),
            pl.BlockSpec((1920, 640), lambda n: (0, 0)),
            pl.BlockSpec((1, 320), lambda n: (0, 0)),
            pl.BlockSpec((16, 320, 128), lambda n: (0, 0, 0)),
            pl.BlockSpec((1, 128), lambda n: (0, 0)),
            pl.BlockSpec((128, n_out), lambda n: (0, 0)),
            pl.BlockSpec((1, n_out), lambda n: (0, 0)),
        ],
        out_specs=pl.BlockSpec((B, n_out), lambda n: (n, 0)),
        scratch_shapes=[
            pltpu.VMEM((4, Bh, 16, 340), _BF16),
            pltpu.VMEM((2, Bh, 16, 1920), _BF16),
            pltpu.VMEM((4, Bh, 16, 340), _BF16),
            pltpu.VMEM((2, Bh, 16, 1920), _BF16),
        ],
        compiler_params=pltpu.CompilerParams(
            dimension_semantics=("parallel",)),
    )(x4, t1m, b1c, t2m, b2v, fc1_ws, fc1_b, fc2_wb, fc2_b)


# speedup vs baseline: 1.1441x; 1.0634x over previous
import jax
import jax.numpy as jnp
from jax.experimental import pallas as pl
from jax.experimental.pallas import tpu as pltpu

_BF16 = jnp.bfloat16
_F32 = jnp.float32


def _stage_a1(x_ref, a1_ref, lo, Bh):
    for r in range(4):
        for di in range(5):
            q, o = (r + di) % 4, (r + di) // 4
            a1_ref[r, :, :, 68 * di:68 * di + 68] = \
                x_ref[lo:lo + Bh, o:o + 16, 68 * q:68 * q + 68]


def _dot1(a1_ref, t1_ref, Bh):
    return jnp.dot(a1_ref[...].reshape(4 * Bh * 16, 340), t1_ref[...],
                   preferred_element_type=_F32).reshape(4, Bh, 16, 768)


def _pool_and_stage_a2(y1, b1_ref, a2_ref, Bh):
    b1v = b1_ref[...]
    pe = jnp.maximum(jnp.maximum(y1[0, :, :, :384], y1[0, :, :, 384:]),
                     jnp.maximum(y1[1, :, :, :384], y1[1, :, :, 384:]))
    po = jnp.maximum(jnp.maximum(y1[2, :, :, :384], y1[2, :, :, 384:]),
                     jnp.maximum(y1[3, :, :, :384], y1[3, :, :, 384:]))
    pe = jnp.maximum(pe + b1v, 0.0).astype(_BF16)
    po = jnp.maximum(po + b1v, 0.0).astype(_BF16)

    zrow = jnp.zeros((Bh, 1, 384), _BF16)
    shifted = {
        (0, 0): jnp.concatenate([zrow, pe[:, 0:15, :]], axis=1),
        (0, 1): pe,
        (0, 2): jnp.concatenate([pe[:, 1:16, :], zrow], axis=1),
        (1, 0): jnp.concatenate([zrow, po[:, 0:15, :]], axis=1),
        (1, 1): po,
        (1, 2): jnp.concatenate([po[:, 1:16, :], zrow], axis=1),
    }
    for s in range(2):
        for di in range(5):
            par, off = (s + di) % 2, (s + di) // 2
            a2_ref[s, :, :, 384 * di:384 * di + 384] = shifted[(par, off)]


def _dot2_pool(a2_ref, t2_ref, b2_ref, Bh):
    y2 = jnp.dot(a2_ref[...].reshape(2 * Bh * 16, 1920), t2_ref[...],
                 preferred_element_type=_F32).reshape(2, Bh, 16, 640)
    b2v = b2_ref[...]
    f = jnp.maximum(jnp.maximum(y2[0, :, :, :320], y2[0, :, :, 320:]),
                    jnp.maximum(y2[1, :, :, :320], y2[1, :, :, 320:]))
    return jnp.maximum(f + b2v, 0.0).astype(_BF16)


def _net_kernel(x_ref, t1_ref, b1_ref, t2_ref, b2_ref,
                w1f_ref, b1f_ref, w2f_ref, b2f_ref, o_ref,
                a1_ref0, a2_ref0, a1_ref1, a2_ref1):
    B = x_ref.shape[0]
    Bh = a1_ref0.shape[1]
    if B > Bh:
        _stage_a1(x_ref, a1_ref0, 0, Bh)
        y1_0 = _dot1(a1_ref0, t1_ref, Bh)
        _stage_a1(x_ref, a1_ref1, Bh, Bh)
        y1_1 = _dot1(a1_ref1, t1_ref, Bh)
        _pool_and_stage_a2(y1_0, b1_ref, a2_ref0, Bh)
        f0 = _dot2_pool(a2_ref0, t2_ref, b2_ref, Bh)
        _pool_and_stage_a2(y1_1, b1_ref, a2_ref1, Bh)
        f1 = _dot2_pool(a2_ref1, t2_ref, b2_ref, Bh)
        f = jnp.concatenate([f0, f1], axis=0)
    else:
        _stage_a1(x_ref, a1_ref0, 0, Bh)
        y1_0 = _dot1(a1_ref0, t1_ref, Bh)
        _pool_and_stage_a2(y1_0, b1_ref, a2_ref0, Bh)
        f = _dot2_pool(a2_ref0, t2_ref, b2_ref, Bh)

    parts = [jnp.dot(f[:, i, :], w1f_ref[i], preferred_element_type=_F32)
             for i in range(16)]
    while len(parts) > 1:
        parts = [parts[j] + parts[j + 1] for j in range(0, len(parts), 2)]
    hacc = jnp.maximum(parts[0] + b1f_ref[...], 0.0).astype(_BF16)
    o_ref[...] = jnp.dot(hacc, w2f_ref[...],
                         preferred_element_type=_F32) + b2f_ref[...]


def _build_t1(w1, b1):
    w1r = w1.reshape(10, 5, 5)
    d5 = (jnp.arange(68)[None, :, None]
          == jnp.arange(64)[None, None, :] + jnp.arange(5)[:, None, None])
    t1 = jnp.einsum('cie,etj->itcj', w1r, d5.astype(_F32))
    t1e = jnp.pad(t1[..., 0::2], ((0, 0), (0, 0), (0, 0), (2, 2)))
    t1o = jnp.pad(t1[..., 1::2], ((0, 0), (0, 0), (0, 0), (2, 2)))
    t1e = jnp.pad(t1e.reshape(340, 360), ((0, 0), (0, 24)))
    t1o = jnp.pad(t1o.reshape(340, 360), ((0, 0), (0, 24)))
    t1m = jnp.concatenate([t1e, t1o], axis=1).astype(_BF16)
    b1c = jnp.pad(jnp.broadcast_to(b1, (10, 32)),
                  ((0, 0), (2, 2))).reshape(1, 360)
    b1c = jnp.pad(b1c, ((0, 0), (0, 24)))
    return t1m, b1c


def _build_t2(w2, b2):
    w2r = w2.reshape(20, 5, 5, 10)
    d5 = (jnp.arange(36)[None, :, None]
          == jnp.arange(32)[None, None, :] + jnp.arange(5)[:, None, None])
    t2 = jnp.einsum('diec,etj->ictdj', w2r, d5.astype(_F32))
    t2e = jnp.pad(t2[..., 0::2].reshape(5, 360, 320), ((0, 0), (0, 24), (0, 0)))
    t2o = jnp.pad(t2[..., 1::2].reshape(5, 360, 320), ((0, 0), (0, 24), (0, 0)))
    t2m = jnp.concatenate([t2e.reshape(1920, 320),
                           t2o.reshape(1920, 320)],
                          axis=1).astype(_BF16)
    b2v = jnp.broadcast_to(b2, (20, 16)).reshape(1, 320)
    return t2m, b2v


def kernel(x, w1, b1, w2, b2, fc1_w, fc1_b, fc2_w, fc2_b):
    N = x.shape[0]
    B = 64
    while N % B:
        B //= 2
    Bh = max(B // 2, 1)

    xpad = jnp.pad(x[:, 0], ((0, 0), (2, 2), (2, 2)))
    x4 = xpad.reshape(N, 17, 272).astype(_BF16)

    t1m, b1c = _build_t1(w1, b1)
    t2m, b2v = _build_t2(w2, b2)
    fc1_ws = fc1_w.reshape(20, 16, 16, 128).transpose(1, 0, 2, 3) \
        .reshape(16, 320, 128).astype(_BF16)
    fc2_wb = fc2_w.astype(_BF16)
    n_out = fc2_w.shape[1]

    return pl.pallas_call(
        _net_kernel,
        out_shape=jax.ShapeDtypeStruct((N, n_out), _F32),
        grid=(N // B,),
        in_specs=[
            pl.BlockSpec((B, 17, 272), lambda n: (n, 0, 0)),
            pl.BlockSpec((340, 768), lambda n: (0, 0)),
            pl.BlockSpec((1, 384), lambda n: (0, 0)),
            pl.BlockSpec((1920, 640), lambda n: (0, 0)),
            pl.BlockSpec((1, 320), lambda n: (0, 0)),
            pl.BlockSpec((16, 320, 128), lambda n: (0, 0, 0)),
            pl.BlockSpec((1, 128), lambda n: (0, 0)),
            pl.BlockSpec((128, n_out), lambda n: (0, 0)),
            pl.BlockSpec((1, n_out), lambda n: (0, 0)),
        ],
        out_specs=pl.BlockSpec((B, n_out), lambda n: (n, 0)),
        scratch_shapes=[
            pltpu.VMEM((4, Bh, 16, 340), _BF16),
            pltpu.VMEM((2, Bh, 16, 1920), _BF16),
            pltpu.VMEM((4, Bh, 16, 340), _BF16),
            pltpu.VMEM((2, Bh, 16, 1920), _BF16),
        ],
        compiler_params=pltpu.CompilerParams(
            dimension_semantics=("parallel",)),
    )(x4, t1m, b1c, t2m, b2v, fc1_ws, fc1_b, fc2_wb, fc2_b)


# speedup vs baseline: 1.1546x; 1.0092x over previous
import jax
import jax.numpy as jnp
from jax.experimental import pallas as pl
from jax.experimental.pallas import tpu as pltpu

_BF16 = jnp.bfloat16
_F32 = jnp.float32


def _stage_a1(x_ref, a1_ref, lo, Bh):
    for r in range(4):
        for di in range(5):
            q, o = (r + di) % 4, (r + di) // 4
            a1_ref[r, :, :, 68 * di:68 * di + 68] = \
                x_ref[lo:lo + Bh, o:o + 16, 68 * q:68 * q + 68]


def _dot1(a1_ref, t1_ref, Bh):
    return jnp.dot(a1_ref[...].reshape(4 * Bh * 16, 340), t1_ref[...],
                   preferred_element_type=_F32).reshape(4, Bh, 16, 768)


def _pool_and_stage_a2(y1, b1_ref, a2_ref, Bh):
    b1v = b1_ref[...]
    pe = jnp.maximum(jnp.maximum(y1[0, :, :, :384], y1[0, :, :, 384:]),
                     jnp.maximum(y1[1, :, :, :384], y1[1, :, :, 384:]))
    po = jnp.maximum(jnp.maximum(y1[2, :, :, :384], y1[2, :, :, 384:]),
                     jnp.maximum(y1[3, :, :, :384], y1[3, :, :, 384:]))
    pe = jnp.maximum(pe + b1v, 0.0).astype(_BF16)
    po = jnp.maximum(po + b1v, 0.0).astype(_BF16)

    zrow = jnp.zeros((Bh, 1, 384), _BF16)
    views = [
        jnp.concatenate([zrow, pe[:, 0:15, :]], axis=1),
        jnp.concatenate([zrow, po[:, 0:15, :]], axis=1),
        pe,
        po,
        jnp.concatenate([pe[:, 1:16, :], zrow], axis=1),
        jnp.concatenate([po[:, 1:16, :], zrow], axis=1),
    ]
    for blk, v in enumerate(views):
        a2_ref[:, :, 384 * blk:384 * blk + 384] = v


def _dot2_pool(a2_ref, t2_ref, b2_ref, Bh):
    b2v = b2_ref[...]
    y2 = [jnp.dot(a2_ref[:, :, 384 * s:384 * s + 1920].reshape(Bh * 16, 1920),
                  t2_ref[...],
                  preferred_element_type=_F32).reshape(Bh, 16, 640)
          for s in range(2)]
    f = jnp.maximum(jnp.maximum(y2[0][:, :, :320], y2[0][:, :, 320:]),
                    jnp.maximum(y2[1][:, :, :320], y2[1][:, :, 320:]))
    return jnp.maximum(f + b2v, 0.0).astype(_BF16)


def _net_kernel(x_ref, t1_ref, b1_ref, t2_ref, b2_ref,
                w1f_ref, b1f_ref, w2f_ref, b2f_ref, o_ref,
                a1_ref0, a2_ref0, a1_ref1, a2_ref1):
    B = x_ref.shape[0]
    Bh = a1_ref0.shape[1]
    if B > Bh:
        _stage_a1(x_ref, a1_ref0, 0, Bh)
        y1_0 = _dot1(a1_ref0, t1_ref, Bh)
        _stage_a1(x_ref, a1_ref1, Bh, Bh)
        y1_1 = _dot1(a1_ref1, t1_ref, Bh)
        _pool_and_stage_a2(y1_0, b1_ref, a2_ref0, Bh)
        f0 = _dot2_pool(a2_ref0, t2_ref, b2_ref, Bh)
        _pool_and_stage_a2(y1_1, b1_ref, a2_ref1, Bh)
        f1 = _dot2_pool(a2_ref1, t2_ref, b2_ref, Bh)
        f = jnp.concatenate([f0, f1], axis=0)
    else:
        _stage_a1(x_ref, a1_ref0, 0, Bh)
        y1_0 = _dot1(a1_ref0, t1_ref, Bh)
        _pool_and_stage_a2(y1_0, b1_ref, a2_ref0, Bh)
        f = _dot2_pool(a2_ref0, t2_ref, b2_ref, Bh)

    parts = [jnp.dot(f[:, i, :], w1f_ref[i], preferred_element_type=_F32)
             for i in range(16)]
    while len(parts) > 1:
        parts = [parts[j] + parts[j + 1] for j in range(0, len(parts), 2)]
    hacc = jnp.maximum(parts[0] + b1f_ref[...], 0.0).astype(_BF16)
    o_ref[...] = jnp.dot(hacc, w2f_ref[...],
                         preferred_element_type=_F32) + b2f_ref[...]


def _build_t1(w1, b1):
    w1r = w1.reshape(10, 5, 5)
    d5 = (jnp.arange(68)[None, :, None]
          == jnp.arange(64)[None, None, :] + jnp.arange(5)[:, None, None])
    t1 = jnp.einsum('cie,etj->itcj', w1r, d5.astype(_F32))
    t1e = jnp.pad(t1[..., 0::2], ((0, 0), (0, 0), (0, 0), (2, 2)))
    t1o = jnp.pad(t1[..., 1::2], ((0, 0), (0, 0), (0, 0), (2, 2)))
    t1e = jnp.pad(t1e.reshape(340, 360), ((0, 0), (0, 24)))
    t1o = jnp.pad(t1o.reshape(340, 360), ((0, 0), (0, 24)))
    t1m = jnp.concatenate([t1e, t1o], axis=1).astype(_BF16)
    b1c = jnp.pad(jnp.broadcast_to(b1, (10, 32)),
                  ((0, 0), (2, 2))).reshape(1, 360)
    b1c = jnp.pad(b1c, ((0, 0), (0, 24)))
    return t1m, b1c


def _build_t2(w2, b2):
    w2r = w2.reshape(20, 5, 5, 10)
    d5 = (jnp.arange(36)[None, :, None]
          == jnp.arange(32)[None, None, :] + jnp.arange(5)[:, None, None])
    t2 = jnp.einsum('diec,etj->ictdj', w2r, d5.astype(_F32))
    t2e = jnp.pad(t2[..., 0::2].reshape(5, 360, 320), ((0, 0), (0, 24), (0, 0)))
    t2o = jnp.pad(t2[..., 1::2].reshape(5, 360, 320), ((0, 0), (0, 24), (0, 0)))
    t2m = jnp.concatenate([t2e.reshape(1920, 320),
                           t2o.reshape(1920, 320)],
                          axis=1).astype(_BF16)
    b2v = jnp.broadcast_to(b2, (20, 16)).reshape(1, 320)
    return t2m, b2v


def kernel(x, w1, b1, w2, b2, fc1_w, fc1_b, fc2_w, fc2_b):
    N = x.shape[0]
    B = 64
    while N % B:
        B //= 2
    Bh = max(B // 2, 1)

    xpad = jnp.pad(x[:, 0], ((0, 0), (2, 2), (2, 2)))
    x4 = xpad.reshape(N, 17, 272).astype(_BF16)

    t1m, b1c = _build_t1(w1, b1)
    t2m, b2v = _build_t2(w2, b2)
    fc1_ws = fc1_w.reshape(20, 16, 16, 128).transpose(1, 0, 2, 3) \
        .reshape(16, 320, 128).astype(_BF16)
    fc2_wb = fc2_w.astype(_BF16)
    n_out = fc2_w.shape[1]

    return pl.pallas_call(
        _net_kernel,
        out_shape=jax.ShapeDtypeStruct((N, n_out), _F32),
        grid=(N // B,),
        in_specs=[
            pl.BlockSpec((B, 17, 272), lambda n: (n, 0, 0)),
            pl.BlockSpec((340, 768), lambda n: (0, 0)),
            pl.BlockSpec((1, 384), lambda n: (0, 0)),
            pl.BlockSpec((1920, 640), lambda n: (0, 0)),
            pl.BlockSpec((1, 320), lambda n: (0, 0)),
            pl.BlockSpec((16, 320, 128), lambda n: (0, 0, 0)),
            pl.BlockSpec((1, 128), lambda n: (0, 0)),
            pl.BlockSpec((128, n_out), lambda n: (0, 0)),
            pl.BlockSpec((1, n_out), lambda n: (0, 0)),
        ],
        out_specs=pl.BlockSpec((B, n_out), lambda n: (n, 0)),
        scratch_shapes=[
            pltpu.VMEM((4, Bh, 16, 340), _BF16),
            pltpu.VMEM((Bh, 16, 2304), _BF16),
            pltpu.VMEM((4, Bh, 16, 340), _BF16),
            pltpu.VMEM((Bh, 16, 2304), _BF16),
        ],
        compiler_params=pltpu.CompilerParams(
            dimension_semantics=("parallel",)),
    )(x4, t1m, b1c, t2m, b2v, fc1_ws, fc1_b, fc2_wb, fc2_b)
